# baseline (speedup 1.0000x reference)
import functools

import jax
import jax.numpy as jnp
from jax import lax
from jax.experimental import pallas as pl
from jax.experimental.pallas import tpu as pltpu
from jax.experimental.pallas import tpu_sc as plsc

N_TOKENS = 8192
D_MODEL = 768
PATHS = 8
NC = 2
NS = 16
NW = NC * NS
ROWS_PER_W = N_TOKENS // NW
ZBLK = 128
NZ = ROWS_PER_W // ZBLK


def _sc_dispatch_body(in_hbm, zsrc_hbm, *rest):
    outs = rest[:PATHS]
    zbuf, sem = rest[PATHS], rest[PATHS + 1]
    wid = lax.axis_index("s") * NC + lax.axis_index("c")
    base = wid * ROWS_PER_W
    pltpu.sync_copy(zsrc_hbm, zbuf)
    copies = [
        pltpu.async_copy(
            in_hbm.at[pl.ds(base, ROWS_PER_W)],
            outs[0].at[pl.ds(base, ROWS_PER_W)],
            sem,
        )
    ]
    for p in range(1, PATHS):
        for j in range(NZ):
            copies.append(
                pltpu.async_copy(
                    zbuf, outs[p].at[pl.ds(base + j * ZBLK, ZBLK)], sem
                )
            )
    for c in copies:
        c.wait()


_sc_dispatch = functools.partial(
    pl.kernel,
    mesh=plsc.VectorSubcoreMesh(core_axis_name="c", subcore_axis_name="s"),
    out_type=tuple(
        jax.ShapeDtypeStruct((N_TOKENS, D_MODEL), jnp.float32)
        for _ in range(PATHS)
    ),
    scratch_types=[
        pltpu.VMEM((ZBLK, D_MODEL), jnp.float32),
        pltpu.SemaphoreType.DMA,
    ],
)(_sc_dispatch_body)


def kernel(inputs):
    zsrc = jnp.zeros((ZBLK, D_MODEL), dtype=inputs.dtype)
    return _sc_dispatch(inputs, zsrc)


# speedup vs baseline: 7.4114x; 7.4114x over previous
import functools

import jax
import jax.numpy as jnp
from jax import lax
from jax.experimental import pallas as pl
from jax.experimental.pallas import tpu as pltpu
from jax.experimental.pallas import tpu_sc as plsc

N_TOKENS = 8192
D_MODEL = 768
PATHS = 8
NC = 2
NS = 16
NW = NC * NS
ROWS_PER_W = N_TOKENS // NW
ZBLK = 128
NZ = ROWS_PER_W // ZBLK
CBLK = 16
NCOPY = ROWS_PER_W // CBLK


def _sc_dispatch_body(in_hbm, zsrc_hbm, *rest):
    outs = rest[:PATHS]
    zbuf = rest[PATHS]
    cbufs = rest[PATHS + 1: PATHS + 3]
    sem, gsem = rest[PATHS + 3], rest[PATHS + 4]
    ssems = rest[PATHS + 5: PATHS + 7]
    wid = lax.axis_index("s") * NC + lax.axis_index("c")
    base = wid * ROWS_PER_W
    pltpu.sync_copy(zsrc_hbm, zbuf)
    copies = []
    for p in range(1, PATHS):
        for j in range(NZ):
            copies.append(
                pltpu.async_copy(
                    zbuf, outs[p].at[pl.ds(base + j * ZBLK, ZBLK)], sem
                )
            )
    scatters = [None, None]
    for j in range(NCOPY):
        lo = base + j * CBLK
        b = j % 2
        if scatters[b] is not None:
            scatters[b].wait()
        pltpu.async_copy(in_hbm.at[pl.ds(lo, CBLK)], cbufs[b], gsem).wait()
        scatters[b] = pltpu.async_copy(
            cbufs[b], outs[0].at[pl.ds(lo, CBLK)], ssems[b]
        )
    for c in copies + scatters:
        c.wait()


_sc_dispatch = functools.partial(
    pl.kernel,
    mesh=plsc.VectorSubcoreMesh(core_axis_name="c", subcore_axis_name="s"),
    out_type=tuple(
        jax.ShapeDtypeStruct((N_TOKENS, D_MODEL), jnp.float32)
        for _ in range(PATHS)
    ),
    scratch_types=[
        pltpu.VMEM((ZBLK, D_MODEL), jnp.float32),
        pltpu.VMEM((CBLK, D_MODEL), jnp.float32),
        pltpu.VMEM((CBLK, D_MODEL), jnp.float32),
        pltpu.SemaphoreType.DMA,
        pltpu.SemaphoreType.DMA,
        pltpu.SemaphoreType.DMA,
        pltpu.SemaphoreType.DMA,
    ],
)(_sc_dispatch_body)


def kernel(inputs):
    zsrc = jnp.zeros((ZBLK, D_MODEL), dtype=inputs.dtype)
    return _sc_dispatch(inputs, zsrc)


# speedup vs baseline: 9.1546x; 1.2352x over previous
import functools

import jax
import jax.numpy as jnp
from jax import lax
from jax.experimental import pallas as pl
from jax.experimental.pallas import tpu as pltpu
from jax.experimental.pallas import tpu_sc as plsc

N_TOKENS = 8192
D_MODEL = 768
PATHS = 8
NC = 2
NS = 16
NW = NC * NS
ROWS_PER_W = N_TOKENS // NW
CBLK = 64
NCOPY = ROWS_PER_W // CBLK
ZBLOCK = 1024


def _sc_copy_body(in_hbm, out0, b0, b1, gsem, s0, s1):
    cbufs = (b0, b1)
    ssems = (s0, s1)
    wid = lax.axis_index("s") * NC + lax.axis_index("c")
    base = wid * ROWS_PER_W
    scatters = [None, None]
    for j in range(NCOPY):
        lo = base + j * CBLK
        b = j % 2
        if scatters[b] is not None:
            scatters[b].wait()
        pltpu.async_copy(in_hbm.at[pl.ds(lo, CBLK)], cbufs[b], gsem).wait()
        scatters[b] = pltpu.async_copy(
            cbufs[b], out0.at[pl.ds(lo, CBLK)], ssems[b]
        )
    for c in scatters:
        if c is not None:
            c.wait()


_sc_copy = functools.partial(
    pl.kernel,
    mesh=plsc.VectorSubcoreMesh(core_axis_name="c", subcore_axis_name="s"),
    out_type=jax.ShapeDtypeStruct((N_TOKENS, D_MODEL), jnp.float32),
    scratch_types=[
        pltpu.VMEM((CBLK, D_MODEL), jnp.float32),
        pltpu.VMEM((CBLK, D_MODEL), jnp.float32),
        pltpu.SemaphoreType.DMA,
        pltpu.SemaphoreType.DMA,
        pltpu.SemaphoreType.DMA,
    ],
)(_sc_copy_body)


def _tc_zero_body(*out_refs):
    for r in out_refs:
        r[...] = jnp.zeros_like(r)


def _tc_zeros(n, d, dtype):
    spec = pl.BlockSpec((ZBLOCK, d), lambda i: (i, 0))
    return pl.pallas_call(
        _tc_zero_body,
        grid=(n // ZBLOCK,),
        in_specs=[],
        out_specs=tuple(spec for _ in range(PATHS - 1)),
        out_shape=tuple(
            jax.ShapeDtypeStruct((n, d), dtype) for _ in range(PATHS - 1)
        ),
    )()


def kernel(inputs):
    n, d = inputs.shape
    out0 = _sc_copy(inputs)
    zeros = _tc_zeros(n, d, inputs.dtype)
    return (out0,) + tuple(zeros)
